# baseline (speedup 1.0000x reference)
import functools

import jax
import jax.numpy as jnp
from jax import lax
from jax.experimental import pallas as pl
from jax.experimental.pallas import tpu as pltpu
from jax.experimental.pallas import tpu_sc as plsc

B = 20; M = 500; N = B * M; E = 320000; LQ = 20; H = 128
REL = 1000; RDIM = 128; EDIM = 128; T = 3

NC = 2
NS = 16
W = H // NS
EHALF = E // NC
CHUNK = 4096
GPAD = 10240
GPW = GPAD // (NC * NS)

_sc_mesh = functools.partial(
    plsc.VectorSubcoreMesh, core_axis_name="c", subcore_axis_name="s",
    num_cores=NC, num_subcores=NS)


def _gather_body(tbl_hbm, idx_hbm, out_hbm, idx_v, rows_v, sem):
    cid = lax.axis_index("c")
    sid = lax.axis_index("s")
    wid = sid * NC + cid
    base = wid * GPW
    pltpu.sync_copy(idx_hbm.at[pl.ds(base, GPW)], idx_v)
    pltpu.async_copy(tbl_hbm.at[idx_v], rows_v, sem).wait()
    pltpu.sync_copy(rows_v, out_hbm.at[pl.ds(base, GPW)])


def _ent_gather(tbl, idx):
    f = pl.kernel(
        _gather_body,
        out_type=jax.ShapeDtypeStruct((GPAD, EDIM), jnp.float32),
        mesh=_sc_mesh(),
        scratch_types=[
            pltpu.VMEM((GPW,), jnp.int32),
            pltpu.VMEM((GPW, EDIM), jnp.float32),
            pltpu.SemaphoreType.DMA,
        ],
        compiler_params=pltpu.CompilerParams(needs_layout_passes=False),
    )
    return f(tbl, idx)


def _edge_body(pk1_hbm, pk2_hbm, tl_hbm, rp_hbm, ins_hbm, rng_hbm, out_hbm,
               rp_v, combo_v, ins_v, tl_v, rng_v, neigh_v,
               pk1a, pk1b, pk2a, pk2b, sem0, sem1):
    cid = lax.axis_index("c")
    sid = lax.axis_index("s")
    fbase = sid * W

    pltpu.sync_copy(rp_hbm.at[pl.ds(fbase, W), :], rp_v)
    pltpu.sync_copy(ins_hbm.at[pl.ds(fbase, W), :], ins_v)
    pltpu.sync_copy(tl_hbm, tl_v)
    pltpu.sync_copy(rng_hbm, rng_v)

    zeros16 = jnp.zeros((16,), jnp.float32)
    for w in range(W):
        @plsc.parallel_loop(0, N // 16, 1, unroll=8)
        def _zbody(i, w=w):
            neigh_v[w, pl.ds(i * 16, 16)] = zeros16

    ebase = cid * EHALF
    eend = ebase + EHALF
    sems = (sem0, sem1)
    bufs1 = (pk1a, pk1b)
    bufs2 = (pk2a, pk2b)
    wvecs = [jnp.full((16,), w, jnp.int32) for w in range(W)]
    lanes = jnp.arange(16, dtype=jnp.int32)
    rv0 = rng_v[pl.ds(0, 16)]
    rv1 = rng_v[pl.ds(16, 16)]

    def _range_at(b):
        lo = lax.reduce_max(jnp.where(lanes == b, rv0, 0), (0,))
        hi = lax.reduce_max(jnp.where(lanes == b - 16, rv1, 0), (0,))
        return lo + hi

    def _issue(start, c, buf):
        off = pl.multiple_of(start + c * CHUNK, 16)
        pltpu.async_copy(pk1_hbm.at[pl.ds(off, CHUNK)], bufs1[buf], sems[buf])
        pltpu.async_copy(pk2_hbm.at[pl.ds(off, CHUNK)], bufs2[buf], sems[buf])

    def _wait(buf):
        pltpu.make_async_copy(pk1_hbm.at[pl.ds(0, CHUNK)],
                              bufs1[buf], sems[buf]).wait()
        pltpu.make_async_copy(pk2_hbm.at[pl.ds(0, CHUNK)],
                              bufs2[buf], sems[buf]).wait()

    def _process(buf, cbase, lo, hi):
        @plsc.parallel_loop(0, CHUNK // 16, 1, unroll=8)
        def _gbody(j):
            a = bufs1[buf][pl.ds(j * 16, 16)]
            tail = bufs2[buf][pl.ds(j * 16, 16)]
            ge = cbase + j * 16 + lanes
            m = jnp.logical_and(ge >= lo, ge < hi)
            rel = a & 1023
            head = a >> 10
            tlv = plsc.load_gather(tl_v, [head])
            for wp in range(W // 2):
                cvp = plsc.load_gather(combo_v, [wvecs[wp], rel])
                v0, v1 = plsc.unpack(
                    plsc.bitcast(cvp, jnp.bfloat16),
                    format=plsc.PackFormat.INTERLEAVED,
                    preferred_element_type=jnp.float32)
                plsc.addupdate_scatter(neigh_v, [wvecs[2 * wp], tail],
                                       v0 * tlv, mask=m)
                plsc.addupdate_scatter(neigh_v, [wvecs[2 * wp + 1], tail],
                                       v1 * tlv, mask=m)

    def _batch(b, _):
        lo = jnp.clip(_range_at(b), ebase, eend)
        hi = jnp.clip(_range_at(b + 1), ebase, eend)
        start = lo & ~15
        nch = (hi - start + CHUNK - 1) >> 12

        @pl.when(nch > 0)
        def _():
            _issue(start, 0, 0)
        bvec = jnp.full((16,), b, jnp.int32)
        insb = [plsc.load_gather(ins_v, [wvecs[w], bvec]) for w in range(W)]

        def _cstep(i):
            for wp in range(W // 2):
                v0 = rp_v[2 * wp, pl.ds(i * 16, 16)] * insb[2 * wp]
                v1 = rp_v[2 * wp + 1, pl.ds(i * 16, 16)] * insb[2 * wp + 1]
                pkd = plsc.pack(v0, v1, format=plsc.PackFormat.INTERLEAVED)
                combo_v[wp, pl.ds(i * 16, 16)] = plsc.bitcast(pkd, jnp.int32)

        @plsc.parallel_loop(0, (REL - 16 + 15) // 16, 1, unroll=4)
        def _cbody(i):
            _cstep(i)

        def _cstep_tail():
            i0 = REL - 16
            for wp in range(W // 2):
                v0 = rp_v[2 * wp, pl.ds(i0, 16)] * insb[2 * wp]
                v1 = rp_v[2 * wp + 1, pl.ds(i0, 16)] * insb[2 * wp + 1]
                pkd = plsc.pack(v0, v1, format=plsc.PackFormat.INTERLEAVED)
                combo_v[wp, pl.ds(i0, 16)] = plsc.bitcast(pkd, jnp.int32)
        _cstep_tail()

        def _pair(p, _):
            c0 = 2 * p
            _wait(0)

            @pl.when(c0 + 1 < nch)
            def _():
                _issue(start, c0 + 1, 1)
            _process(0, start + c0 * CHUNK, lo, hi)

            @pl.when(c0 + 1 < nch)
            def _():
                _wait(1)

                @pl.when(c0 + 2 < nch)
                def _():
                    _issue(start, c0 + 2, 0)
                _process(1, start + (c0 + 1) * CHUNK, lo, hi)
            return 0
        lax.fori_loop(0, (nch + 1) >> 1, _pair, 0)
        return 0

    lax.fori_loop(0, B, _batch, 0)

    pltpu.sync_copy(neigh_v, out_hbm.at[cid, pl.ds(fbase, W), :])


def _edge_sweep(pk1, pk2, tl_flat, rp_fm, ins_fm, rngs):
    f = pl.kernel(
        _edge_body,
        out_type=jax.ShapeDtypeStruct((NC, H, N), jnp.float32),
        mesh=_sc_mesh(),
        scratch_types=[
            pltpu.VMEM((W, REL), jnp.float32),
            pltpu.VMEM((W // 2, REL), jnp.int32),
            pltpu.VMEM((W, B), jnp.float32),
            pltpu.VMEM((N,), jnp.float32),
            pltpu.VMEM((32,), jnp.int32),
            pltpu.VMEM((W, N), jnp.float32),
            pltpu.VMEM((CHUNK,), jnp.int32),
            pltpu.VMEM((CHUNK,), jnp.int32),
            pltpu.VMEM((CHUNK,), jnp.int32),
            pltpu.VMEM((CHUNK,), jnp.int32),
            pltpu.SemaphoreType.DMA,
            pltpu.SemaphoreType.DMA,
        ],
        compiler_params=pltpu.CompilerParams(needs_layout_passes=False),
    )
    return f(pk1, pk2, tl_flat, rp_fm, ins_fm, rngs)


def _tc_a_body(we_ref, q_ref, mb_ref, wx_ref, wh_ref, bl_ref, wq_ref, bq_ref,
               watt_ref, qmask_ref, rel_ref, lng_ref, lnb_ref, wrel_ref,
               brel_ref, rows_ref, went_ref, bent_ref,
               ht_ref, ins_ref, rp_ref, ent0_ref, xs_ref, hs_ref, xz_ref,
               sem):
    f32 = jnp.float32

    def _issue(i, _):
        b = i // LQ
        l = i % LQ
        idx = q_ref[b, l]
        pltpu.make_async_copy(we_ref.at[pl.ds(idx, 1), :],
                              xs_ref.at[l, pl.ds(b, 1), :], sem).start()
        return 0
    lax.fori_loop(0, B * LQ, _issue, 0)

    def _drain(i, _):
        pltpu.make_async_copy(we_ref.at[pl.ds(0, 1), :],
                              xs_ref.at[0, pl.ds(0, 1), :], sem).wait()
        return 0
    lax.fori_loop(0, B * LQ, _drain, 0)

    wh = wh_ref[...]
    bl = bl_ref[...]
    xz_ref[...] = jnp.dot(
        xs_ref[...].reshape(LQ * B, -1), wx_ref[...],
        preferred_element_type=f32).reshape(LQ, B, 4 * H)

    def step(t, carry):
        h, c = carry
        mt = mb_ref[t]
        z = (xz_ref[t]
             + jnp.dot(h, wh, preferred_element_type=f32) + bl[None, :])
        i = jax.nn.sigmoid(z[:, 0:H])
        fg = jax.nn.sigmoid(z[:, H:2 * H])
        g = jnp.tanh(z[:, 2 * H:3 * H])
        o = jax.nn.sigmoid(z[:, 3 * H:4 * H])
        cn = fg * c + i * g
        hn = o * jnp.tanh(cn)
        h2 = mt * hn + (1.0 - mt) * h
        c2 = mt * cn + (1.0 - mt) * c
        hs_ref[t] = h2
        return (h2, c2)

    h0 = jnp.zeros((B, H), f32)
    hT, _ = lax.fori_loop(0, LQ, step, (h0, h0))
    ht_ref[...] = hT

    hs = hs_ref[...]
    qmask = qmask_ref[...]
    for t in range(T):
        q_t = jnp.dot(hT, wq_ref[t], preferred_element_type=f32) + bq_ref[t][None, :]
        prod = hs * q_t[None, :, :] * watt_ref[t][None, None, :]
        logits = jnp.sum(prod, axis=2)
        logits = jnp.transpose(logits)
        logits = jnp.where(qmask > 0, logits, -1e20)
        mx = jnp.max(logits, axis=1, keepdims=True)
        ex = jnp.exp(logits - mx)
        attn = ex / jnp.sum(ex, axis=1, keepdims=True)
        attn_t = jnp.transpose(attn)
        ins_t = jnp.sum(attn_t[:, :, None] * hs, axis=0)
        ins_ref[t] = jnp.transpose(ins_t)

    rel = rel_ref[...]
    mu = jnp.mean(rel, axis=1, keepdims=True)
    var = jnp.mean((rel - mu) ** 2, axis=1, keepdims=True)
    ln = (rel - mu) / jnp.sqrt(var + 1e-5) * lng_ref[...][None, :] + lnb_ref[...][None, :]
    for t in range(T):
        rp_t = jnp.maximum(
            jnp.dot(ln, wrel_ref[t], preferred_element_type=f32) + brel_ref[t][None, :],
            0.0)
        rp_ref[t] = jnp.transpose(rp_t)

    ent0 = lax.dot_general(went_ref[...], rows_ref[...],
                           (((0,), (1,)), ((), ())),
                           preferred_element_type=f32)
    ent0_ref[...] = ent0 + bent_ref[...][:, None]


def _tc_a(question, mb, qmask, params, rows):
    out_shape = [
        jax.ShapeDtypeStruct((B, H), jnp.float32),
        jax.ShapeDtypeStruct((T, H, B), jnp.float32),
        jax.ShapeDtypeStruct((T, H, REL), jnp.float32),
        jax.ShapeDtypeStruct((H, N), jnp.float32),
    ]
    wdim = params["word_emb"].shape[1]
    nin = 18
    in_specs = [pl.BlockSpec(memory_space=pl.ANY),
                pl.BlockSpec(memory_space=pltpu.SMEM)]
    in_specs += [pl.BlockSpec(memory_space=pltpu.VMEM) for _ in range(nin - 2)]
    return pl.pallas_call(
        _tc_a_body,
        out_shape=out_shape,
        in_specs=in_specs,
        scratch_shapes=[pltpu.VMEM((LQ, B, wdim), jnp.float32),
                        pltpu.VMEM((LQ, B, H), jnp.float32),
                        pltpu.VMEM((LQ, B, 4 * H), jnp.float32),
                        pltpu.SemaphoreType.DMA],
    )(params["word_emb"], question, mb, params["Wx"], params["Wh"],
      params["b_lstm"], params["Wq"], params["bq"], params["w_att"], qmask,
      params["rel_emb"], params["ln_g"], params["ln_b"], params["W_rel"],
      params["b_rel"], rows, params["W_ent"], params["b_ent"])


def _tc_cpre_body(ent_ref, w1_ref, be_ref, z1_ref):
    z1_ref[...] = (lax.dot_general(w1_ref[...], ent_ref[...],
                                   (((0,), (0,)), ((), ())),
                                   preferred_element_type=jnp.float32)
                   + be_ref[...][:, None])


def _tc_cpre(ent_fm, w1, be):
    return pl.pallas_call(
        _tc_cpre_body,
        out_shape=jax.ShapeDtypeStruct((H, N), jnp.float32),
    )(ent_fm, w1, be)


def _node_update(z1, neigh_ref, ent_ref, w2_ref):
    nf = neigh_ref[0] + neigh_ref[1]
    z = z1 + lax.dot_general(w2_ref[...], nf, (((0,), (0,)), ((), ())),
                             preferred_element_type=jnp.float32)
    return jnp.maximum(z, 0.0) + ent_ref[...]


def _tc_cmid_body(z1_ref, neigh_ref, ent_ref, w2_ref, ws_ref, bs_ref,
                  mask_ref, entnew_ref, tl_ref):
    ent_new = _node_update(z1_ref[...], neigh_ref, ent_ref, w2_ref)
    entnew_ref[...] = ent_new
    s = jnp.sum(ent_new * ws_ref[...], axis=0, keepdims=True)
    s = s + bs_ref[...][:, None]
    rows = [s[0:1, b * M:(b + 1) * M] for b in range(B)]
    sb = jnp.concatenate(rows, axis=0)
    sb = jnp.where(mask_ref[...] > 0, sb, -1e20)
    mx = jnp.max(sb, axis=1, keepdims=True)
    ex = jnp.exp(sb - mx)
    tl_ref[...] = ex / jnp.sum(ex, axis=1, keepdims=True)


def _tc_cmid(z1, neigh2, ent_fm, w2, ws, bs, entity_mask):
    out_shape = [
        jax.ShapeDtypeStruct((H, N), jnp.float32),
        jax.ShapeDtypeStruct((B, M), jnp.float32),
    ]
    return pl.pallas_call(_tc_cmid_body, out_shape=out_shape)(
        z1, neigh2, ent_fm, w2, ws, bs, entity_mask)


def _tc_clast_body(z1_ref, neigh_ref, ent_ref, w2_ref, wp_ref, bp_ref,
                   ht_ref, mask_ref, out_ref):
    f32 = jnp.float32
    ent_new = _node_update(z1_ref[...], neigh_ref, ent_ref, w2_ref)
    ep = (lax.dot_general(wp_ref[...], ent_new, (((0,), (0,)), ((), ())),
                          preferred_element_type=f32)
          + bp_ref[...][:, None])
    big = jnp.dot(ht_ref[...], ep, preferred_element_type=f32)
    rows = [big[b:b + 1, b * M:(b + 1) * M] for b in range(B)]
    sc = jnp.concatenate(rows, axis=0)
    mask = mask_ref[...]
    out_ref[...] = mask * sc + (1.0 - mask) * -1e20


def _tc_clast(z1, neigh2, ent_fm, w2, wp, bp, ht, entity_mask):
    return pl.pallas_call(
        _tc_clast_body,
        out_shape=jax.ShapeDtypeStruct((B, M), jnp.float32),
    )(z1, neigh2, ent_fm, w2, wp, bp, ht, entity_mask)


def kernel(question, question_mask, topic_label, candidate_entity, entity_mask,
           batch_ids, batch_relations, edge_index, params):
    f32 = jnp.float32
    mb = jnp.swapaxes(question_mask, 0, 1)[:, :, None]

    idx = jnp.concatenate(
        [candidate_entity.reshape(-1).astype(jnp.int32),
         jnp.arange(GPAD - N, dtype=jnp.int32)])
    rows = _ent_gather(params["ent_emb"], idx)[:N]

    head = edge_index[0].astype(jnp.int32)
    tail = edge_index[1].astype(jnp.int32)
    zpad = jnp.zeros((CHUNK,), jnp.int32)
    pk1 = jnp.concatenate(
        [batch_relations.astype(jnp.int32) | (head << 10), zpad])
    pk2 = jnp.concatenate([tail, zpad])
    rngs = jnp.zeros((32,), jnp.int32).at[:B + 1].set(
        jnp.searchsorted(batch_ids, jnp.arange(B + 1)).astype(jnp.int32))

    hT, ins_fm, rp_fm, ent_fm = _tc_a(question.astype(jnp.int32), mb,
                                      question_mask, params, rows)

    tl = topic_label.reshape(-1).astype(f32)
    for t in range(T):
        neigh2 = _edge_sweep(pk1, pk2, tl, rp_fm[t], ins_fm[t], rngs)
        z1 = _tc_cpre(ent_fm, params["W_e"][t][:H], params["b_e"][t])
        if t + 1 < T:
            ent_fm, tl_bm = _tc_cmid(z1, neigh2, ent_fm,
                                     params["W_e"][t][H:],
                                     params["w_score"][t],
                                     params["b_score"][t], entity_mask)
            tl = tl_bm.reshape(-1)
        else:
            return _tc_clast(z1, neigh2, ent_fm, params["W_e"][t][H:],
                             params["W_proj"], params["b_proj"], hT,
                             entity_mask)


# speedup vs baseline: 1.1220x; 1.1220x over previous
import functools

import jax
import jax.numpy as jnp
from jax import lax
from jax.experimental import pallas as pl
from jax.experimental.pallas import tpu as pltpu
from jax.experimental.pallas import tpu_sc as plsc

B = 20; M = 500; N = B * M; E = 320000; LQ = 20; H = 128
REL = 1000; RDIM = 128; EDIM = 128; T = 3

NC = 2
NS = 16
W = H // NS
EHALF = E // NC
CHUNK = 4096
GPAD = 10240
GPW = GPAD // (NC * NS)

_sc_mesh = functools.partial(
    plsc.VectorSubcoreMesh, core_axis_name="c", subcore_axis_name="s",
    num_cores=NC, num_subcores=NS)


def _gather_body(tbl_hbm, idx_hbm, out_hbm, idx_v, rows_v, sem):
    cid = lax.axis_index("c")
    sid = lax.axis_index("s")
    wid = sid * NC + cid
    base = wid * GPW
    pltpu.sync_copy(idx_hbm.at[pl.ds(base, GPW)], idx_v)
    pltpu.async_copy(tbl_hbm.at[idx_v], rows_v, sem).wait()
    pltpu.sync_copy(rows_v, out_hbm.at[pl.ds(base, GPW)])


def _ent_gather(tbl, idx):
    f = pl.kernel(
        _gather_body,
        out_type=jax.ShapeDtypeStruct((GPAD, EDIM), jnp.float32),
        mesh=_sc_mesh(),
        scratch_types=[
            pltpu.VMEM((GPW,), jnp.int32),
            pltpu.VMEM((GPW, EDIM), jnp.float32),
            pltpu.SemaphoreType.DMA,
        ],
        compiler_params=pltpu.CompilerParams(needs_layout_passes=False),
    )
    return f(tbl, idx)


def _edge_body(pk1_hbm, pk2_hbm, tl_hbm, rp_hbm, ins_hbm, rng_hbm, out_hbm,
               rp_v, combo_v, ins_v, tl_v, rng_v, neigh_v,
               pk1a, pk1b, pk2a, pk2b, sem0, sem1):
    cid = lax.axis_index("c")
    sid = lax.axis_index("s")
    fbase = sid * W

    pltpu.sync_copy(rp_hbm.at[pl.ds(fbase, W), :], rp_v)
    pltpu.sync_copy(ins_hbm.at[pl.ds(fbase, W), :], ins_v)
    pltpu.sync_copy(tl_hbm, tl_v)
    pltpu.sync_copy(rng_hbm, rng_v)

    zeros16 = jnp.zeros((16,), jnp.float32)
    for w in range(W):
        @plsc.parallel_loop(0, N // 16, 1, unroll=8)
        def _zbody(i, w=w):
            neigh_v[w, pl.ds(i * 16, 16)] = zeros16

    ebase = cid * EHALF
    eend = ebase + EHALF
    sems = (sem0, sem1)
    bufs1 = (pk1a, pk1b)
    bufs2 = (pk2a, pk2b)
    wvecs = [jnp.full((16,), w, jnp.int32) for w in range(W)]
    lanes = jnp.arange(16, dtype=jnp.int32)
    rv0 = rng_v[pl.ds(0, 16)]
    rv1 = rng_v[pl.ds(16, 16)]

    def _range_at(b):
        lo = lax.reduce_max(jnp.where(lanes == b, rv0, 0), (0,))
        hi = lax.reduce_max(jnp.where(lanes == b - 16, rv1, 0), (0,))
        return lo + hi

    def _issue(start, c, buf):
        off = pl.multiple_of(start + c * CHUNK, 16)
        pltpu.async_copy(pk1_hbm.at[pl.ds(off, CHUNK)], bufs1[buf], sems[buf])
        pltpu.async_copy(pk2_hbm.at[pl.ds(off, CHUNK)], bufs2[buf], sems[buf])

    def _wait(buf):
        pltpu.make_async_copy(pk1_hbm.at[pl.ds(0, CHUNK)],
                              bufs1[buf], sems[buf]).wait()
        pltpu.make_async_copy(pk2_hbm.at[pl.ds(0, CHUNK)],
                              bufs2[buf], sems[buf]).wait()

    def _process(buf, cbase, lo, hi):
        @plsc.parallel_loop(0, CHUNK // 16, 1, unroll=8)
        def _gbody(j):
            a = bufs1[buf][pl.ds(j * 16, 16)]
            tail = bufs2[buf][pl.ds(j * 16, 16)]
            ge = cbase + j * 16 + lanes
            m = jnp.logical_and(ge >= lo, ge < hi)
            rel = a & 1023
            head = a >> 10
            tlv = plsc.load_gather(tl_v, [head])
            for w in range(W):
                cv = plsc.load_gather(combo_v, [wvecs[w], rel])
                plsc.addupdate_scatter(neigh_v, [wvecs[w], tail],
                                       cv * tlv, mask=m)

    def _batch(b, _):
        lo = jnp.clip(_range_at(b), ebase, eend)
        hi = jnp.clip(_range_at(b + 1), ebase, eend)
        start = lo & ~15
        nch = (hi - start + CHUNK - 1) >> 12

        @pl.when(nch > 0)
        def _():
            _issue(start, 0, 0)
        bvec = jnp.full((16,), b, jnp.int32)
        insb = [plsc.load_gather(ins_v, [wvecs[w], bvec]) for w in range(W)]

        def _cstep(i):
            for w in range(W):
                combo_v[w, pl.ds(i * 16, 16)] = (
                    rp_v[w, pl.ds(i * 16, 16)] * insb[w])

        @plsc.parallel_loop(0, (REL - 16 + 15) // 16, 1, unroll=4)
        def _cbody(i):
            _cstep(i)
        _last = REL - 16
        for w in range(W):
            combo_v[w, pl.ds(_last, 16)] = (
                rp_v[w, pl.ds(_last, 16)] * insb[w])

        def _pair(p, _):
            c0 = 2 * p
            _wait(0)

            @pl.when(c0 + 1 < nch)
            def _():
                _issue(start, c0 + 1, 1)
            _process(0, start + c0 * CHUNK, lo, hi)

            @pl.when(c0 + 1 < nch)
            def _():
                _wait(1)

                @pl.when(c0 + 2 < nch)
                def _():
                    _issue(start, c0 + 2, 0)
                _process(1, start + (c0 + 1) * CHUNK, lo, hi)
            return 0
        lax.fori_loop(0, (nch + 1) >> 1, _pair, 0)
        return 0

    lax.fori_loop(0, B, _batch, 0)

    pltpu.sync_copy(neigh_v, out_hbm.at[cid, pl.ds(fbase, W), :])


def _edge_sweep(pk1, pk2, tl_flat, rp_fm, ins_fm, rngs):
    f = pl.kernel(
        _edge_body,
        out_type=jax.ShapeDtypeStruct((NC, H, N), jnp.float32),
        mesh=_sc_mesh(),
        scratch_types=[
            pltpu.VMEM((W, REL), jnp.float32),
            pltpu.VMEM((W, REL), jnp.float32),
            pltpu.VMEM((W, B), jnp.float32),
            pltpu.VMEM((N,), jnp.float32),
            pltpu.VMEM((32,), jnp.int32),
            pltpu.VMEM((W, N), jnp.float32),
            pltpu.VMEM((CHUNK,), jnp.int32),
            pltpu.VMEM((CHUNK,), jnp.int32),
            pltpu.VMEM((CHUNK,), jnp.int32),
            pltpu.VMEM((CHUNK,), jnp.int32),
            pltpu.SemaphoreType.DMA,
            pltpu.SemaphoreType.DMA,
        ],
        compiler_params=pltpu.CompilerParams(needs_layout_passes=False),
    )
    return f(pk1, pk2, tl_flat, rp_fm, ins_fm, rngs)


def _tc_a_body(we_ref, q_ref, mb_ref, wx_ref, wh_ref, bl_ref, wq_ref, bq_ref,
               watt_ref, qmask_ref, rel_ref, lng_ref, lnb_ref, wrel_ref,
               brel_ref, rows_ref, went_ref, bent_ref,
               ht_ref, ins_ref, rp_ref, ent0_ref, xs_ref, hs_ref, xz_ref,
               sem):
    f32 = jnp.float32

    def _issue(i, _):
        b = i // LQ
        l = i % LQ
        idx = q_ref[b, l]
        pltpu.make_async_copy(we_ref.at[pl.ds(idx, 1), :],
                              xs_ref.at[l, pl.ds(b, 1), :], sem).start()
        return 0
    lax.fori_loop(0, B * LQ, _issue, 0)

    def _drain(i, _):
        pltpu.make_async_copy(we_ref.at[pl.ds(0, 1), :],
                              xs_ref.at[0, pl.ds(0, 1), :], sem).wait()
        return 0
    lax.fori_loop(0, B * LQ, _drain, 0)

    wh = wh_ref[...]
    bl = bl_ref[...]
    xz_ref[...] = jnp.dot(
        xs_ref[...].reshape(LQ * B, -1), wx_ref[...],
        preferred_element_type=f32).reshape(LQ, B, 4 * H)

    def step(t, carry):
        h, c = carry
        mt = mb_ref[t]
        z = (xz_ref[t]
             + jnp.dot(h, wh, preferred_element_type=f32) + bl[None, :])
        i = jax.nn.sigmoid(z[:, 0:H])
        fg = jax.nn.sigmoid(z[:, H:2 * H])
        g = jnp.tanh(z[:, 2 * H:3 * H])
        o = jax.nn.sigmoid(z[:, 3 * H:4 * H])
        cn = fg * c + i * g
        hn = o * jnp.tanh(cn)
        h2 = mt * hn + (1.0 - mt) * h
        c2 = mt * cn + (1.0 - mt) * c
        hs_ref[t] = h2
        return (h2, c2)

    h0 = jnp.zeros((B, H), f32)
    hT, _ = lax.fori_loop(0, LQ, step, (h0, h0))
    ht_ref[...] = hT

    hs = hs_ref[...]
    qmask = qmask_ref[...]
    for t in range(T):
        q_t = jnp.dot(hT, wq_ref[t], preferred_element_type=f32) + bq_ref[t][None, :]
        prod = hs * q_t[None, :, :] * watt_ref[t][None, None, :]
        logits = jnp.sum(prod, axis=2)
        logits = jnp.transpose(logits)
        logits = jnp.where(qmask > 0, logits, -1e20)
        mx = jnp.max(logits, axis=1, keepdims=True)
        ex = jnp.exp(logits - mx)
        attn = ex / jnp.sum(ex, axis=1, keepdims=True)
        attn_t = jnp.transpose(attn)
        ins_t = jnp.sum(attn_t[:, :, None] * hs, axis=0)
        ins_ref[t] = jnp.transpose(ins_t)

    rel = rel_ref[...]
    mu = jnp.mean(rel, axis=1, keepdims=True)
    var = jnp.mean((rel - mu) ** 2, axis=1, keepdims=True)
    ln = (rel - mu) / jnp.sqrt(var + 1e-5) * lng_ref[...][None, :] + lnb_ref[...][None, :]
    for t in range(T):
        rp_t = jnp.maximum(
            jnp.dot(ln, wrel_ref[t], preferred_element_type=f32) + brel_ref[t][None, :],
            0.0)
        rp_ref[t] = jnp.transpose(rp_t)

    ent0 = lax.dot_general(went_ref[...], rows_ref[...],
                           (((0,), (1,)), ((), ())),
                           preferred_element_type=f32)
    ent0_ref[...] = ent0 + bent_ref[...][:, None]


def _tc_a(question, mb, qmask, params, rows):
    out_shape = [
        jax.ShapeDtypeStruct((B, H), jnp.float32),
        jax.ShapeDtypeStruct((T, H, B), jnp.float32),
        jax.ShapeDtypeStruct((T, H, REL), jnp.float32),
        jax.ShapeDtypeStruct((H, N), jnp.float32),
    ]
    wdim = params["word_emb"].shape[1]
    nin = 18
    in_specs = [pl.BlockSpec(memory_space=pl.ANY),
                pl.BlockSpec(memory_space=pltpu.SMEM)]
    in_specs += [pl.BlockSpec(memory_space=pltpu.VMEM) for _ in range(nin - 2)]
    return pl.pallas_call(
        _tc_a_body,
        out_shape=out_shape,
        in_specs=in_specs,
        scratch_shapes=[pltpu.VMEM((LQ, B, wdim), jnp.float32),
                        pltpu.VMEM((LQ, B, H), jnp.float32),
                        pltpu.VMEM((LQ, B, 4 * H), jnp.float32),
                        pltpu.SemaphoreType.DMA],
    )(params["word_emb"], question, mb, params["Wx"], params["Wh"],
      params["b_lstm"], params["Wq"], params["bq"], params["w_att"], qmask,
      params["rel_emb"], params["ln_g"], params["ln_b"], params["W_rel"],
      params["b_rel"], rows, params["W_ent"], params["b_ent"])


def _tc_cpre_body(ent_ref, w1_ref, be_ref, z1_ref):
    z1_ref[...] = (lax.dot_general(w1_ref[...], ent_ref[...],
                                   (((0,), (0,)), ((), ())),
                                   preferred_element_type=jnp.float32)
                   + be_ref[...][:, None])


def _tc_cpre(ent_fm, w1, be):
    return pl.pallas_call(
        _tc_cpre_body,
        out_shape=jax.ShapeDtypeStruct((H, N), jnp.float32),
    )(ent_fm, w1, be)


def _node_update(z1, neigh_ref, ent_ref, w2_ref):
    nf = neigh_ref[0] + neigh_ref[1]
    z = z1 + lax.dot_general(w2_ref[...], nf, (((0,), (0,)), ((), ())),
                             preferred_element_type=jnp.float32)
    return jnp.maximum(z, 0.0) + ent_ref[...]


def _tc_cmid_body(z1_ref, neigh_ref, ent_ref, w2_ref, ws_ref, bs_ref,
                  mask_ref, entnew_ref, tl_ref):
    ent_new = _node_update(z1_ref[...], neigh_ref, ent_ref, w2_ref)
    entnew_ref[...] = ent_new
    s = jnp.sum(ent_new * ws_ref[...], axis=0, keepdims=True)
    s = s + bs_ref[...][:, None]
    rows = [s[0:1, b * M:(b + 1) * M] for b in range(B)]
    sb = jnp.concatenate(rows, axis=0)
    sb = jnp.where(mask_ref[...] > 0, sb, -1e20)
    mx = jnp.max(sb, axis=1, keepdims=True)
    ex = jnp.exp(sb - mx)
    tl_ref[...] = ex / jnp.sum(ex, axis=1, keepdims=True)


def _tc_cmid(z1, neigh2, ent_fm, w2, ws, bs, entity_mask):
    out_shape = [
        jax.ShapeDtypeStruct((H, N), jnp.float32),
        jax.ShapeDtypeStruct((B, M), jnp.float32),
    ]
    return pl.pallas_call(_tc_cmid_body, out_shape=out_shape)(
        z1, neigh2, ent_fm, w2, ws, bs, entity_mask)


def _tc_clast_body(z1_ref, neigh_ref, ent_ref, w2_ref, wp_ref, bp_ref,
                   ht_ref, mask_ref, out_ref):
    f32 = jnp.float32
    ent_new = _node_update(z1_ref[...], neigh_ref, ent_ref, w2_ref)
    ep = (lax.dot_general(wp_ref[...], ent_new, (((0,), (0,)), ((), ())),
                          preferred_element_type=f32)
          + bp_ref[...][:, None])
    big = jnp.dot(ht_ref[...], ep, preferred_element_type=f32)
    rows = [big[b:b + 1, b * M:(b + 1) * M] for b in range(B)]
    sc = jnp.concatenate(rows, axis=0)
    mask = mask_ref[...]
    out_ref[...] = mask * sc + (1.0 - mask) * -1e20


def _tc_clast(z1, neigh2, ent_fm, w2, wp, bp, ht, entity_mask):
    return pl.pallas_call(
        _tc_clast_body,
        out_shape=jax.ShapeDtypeStruct((B, M), jnp.float32),
    )(z1, neigh2, ent_fm, w2, wp, bp, ht, entity_mask)


def kernel(question, question_mask, topic_label, candidate_entity, entity_mask,
           batch_ids, batch_relations, edge_index, params):
    f32 = jnp.float32
    mb = jnp.swapaxes(question_mask, 0, 1)[:, :, None]

    idx = jnp.concatenate(
        [candidate_entity.reshape(-1).astype(jnp.int32),
         jnp.arange(GPAD - N, dtype=jnp.int32)])
    rows = _ent_gather(params["ent_emb"], idx)[:N]

    head = edge_index[0].astype(jnp.int32)
    tail = edge_index[1].astype(jnp.int32)
    zpad = jnp.zeros((CHUNK,), jnp.int32)
    pk1 = jnp.concatenate(
        [batch_relations.astype(jnp.int32) | (head << 10), zpad])
    pk2 = jnp.concatenate([tail, zpad])
    rngs = jnp.zeros((32,), jnp.int32).at[:B + 1].set(
        jnp.searchsorted(batch_ids, jnp.arange(B + 1)).astype(jnp.int32))

    hT, ins_fm, rp_fm, ent_fm = _tc_a(question.astype(jnp.int32), mb,
                                      question_mask, params, rows)

    tl = topic_label.reshape(-1).astype(f32)
    for t in range(T):
        neigh2 = _edge_sweep(pk1, pk2, tl, rp_fm[t], ins_fm[t], rngs)
        z1 = _tc_cpre(ent_fm, params["W_e"][t][:H], params["b_e"][t])
        if t + 1 < T:
            ent_fm, tl_bm = _tc_cmid(z1, neigh2, ent_fm,
                                     params["W_e"][t][H:],
                                     params["w_score"][t],
                                     params["b_score"][t], entity_mask)
            tl = tl_bm.reshape(-1)
        else:
            return _tc_clast(z1, neigh2, ent_fm, params["W_e"][t][H:],
                             params["W_proj"], params["b_proj"], hT,
                             entity_mask)


# speedup vs baseline: 1.1912x; 1.0617x over previous
import functools

import jax
import jax.numpy as jnp
from jax import lax
from jax.experimental import pallas as pl
from jax.experimental.pallas import tpu as pltpu
from jax.experimental.pallas import tpu_sc as plsc

B = 20; M = 500; N = B * M; E = 320000; LQ = 20; H = 128
REL = 1000; RDIM = 128; EDIM = 128; T = 3

NC = 2
NS = 16
W = H // NS
EHALF = E // NC
CHUNK = 4096
GPAD = 10240
GPW = GPAD // (NC * NS)

_sc_mesh = functools.partial(
    plsc.VectorSubcoreMesh, core_axis_name="c", subcore_axis_name="s",
    num_cores=NC, num_subcores=NS)


def _gather_body(tbl_hbm, idx_hbm, out_hbm, idx_v, rows_v, sem):
    cid = lax.axis_index("c")
    sid = lax.axis_index("s")
    wid = sid * NC + cid
    base = wid * GPW
    pltpu.sync_copy(idx_hbm.at[pl.ds(base, GPW)], idx_v)
    pltpu.async_copy(tbl_hbm.at[idx_v], rows_v, sem).wait()
    pltpu.sync_copy(rows_v, out_hbm.at[pl.ds(base, GPW)])


def _ent_gather(tbl, idx):
    f = pl.kernel(
        _gather_body,
        out_type=jax.ShapeDtypeStruct((GPAD, EDIM), jnp.float32),
        mesh=_sc_mesh(),
        scratch_types=[
            pltpu.VMEM((GPW,), jnp.int32),
            pltpu.VMEM((GPW, EDIM), jnp.float32),
            pltpu.SemaphoreType.DMA,
        ],
        compiler_params=pltpu.CompilerParams(needs_layout_passes=False),
    )
    return f(tbl, idx)


def _edge_body(pk1_hbm, pk2_hbm, tl_hbm, rp_hbm, ins_hbm, rng_hbm, out_hbm,
               rp_v, combo_v, ins_v, tl_v, rng_v, neigh_v,
               pk1a, pk1b, pk2a, pk2b, sem0, sem1):
    cid = lax.axis_index("c")
    sid = lax.axis_index("s")
    fbase = sid * W

    pltpu.sync_copy(rp_hbm.at[pl.ds(fbase, W), :], rp_v)
    pltpu.sync_copy(ins_hbm.at[pl.ds(fbase, W), :], ins_v)
    pltpu.sync_copy(tl_hbm, tl_v)
    pltpu.sync_copy(rng_hbm, rng_v)

    zeros16 = jnp.zeros((16,), jnp.float32)
    for w in range(W):
        @plsc.parallel_loop(0, N // 16, 1, unroll=8)
        def _zbody(i, w=w):
            neigh_v[w, pl.ds(i * 16, 16)] = zeros16

    ebase = cid * EHALF
    eend = ebase + EHALF
    sems = (sem0, sem1)
    bufs1 = (pk1a, pk1b)
    bufs2 = (pk2a, pk2b)
    wvecs = [jnp.full((16,), w, jnp.int32) for w in range(W)]
    lanes = jnp.arange(16, dtype=jnp.int32)
    rv0 = rng_v[pl.ds(0, 16)]
    rv1 = rng_v[pl.ds(16, 16)]

    def _range_at(b):
        lo = lax.reduce_max(jnp.where(lanes == b, rv0, 0), (0,))
        hi = lax.reduce_max(jnp.where(lanes == b - 16, rv1, 0), (0,))
        return lo + hi

    def _issue(start, c, buf):
        off = pl.multiple_of(start + c * CHUNK, 16)
        pltpu.async_copy(pk1_hbm.at[pl.ds(off, CHUNK)], bufs1[buf], sems[buf])
        pltpu.async_copy(pk2_hbm.at[pl.ds(off, CHUNK)], bufs2[buf], sems[buf])

    def _wait(buf):
        pltpu.make_async_copy(pk1_hbm.at[pl.ds(0, CHUNK)],
                              bufs1[buf], sems[buf]).wait()
        pltpu.make_async_copy(pk2_hbm.at[pl.ds(0, CHUNK)],
                              bufs2[buf], sems[buf]).wait()

    def _process(buf, cbase, lo, hi):
        @plsc.parallel_loop(0, CHUNK // 16, 1, unroll=8)
        def _gbody(j):
            a = bufs1[buf][pl.ds(j * 16, 16)]
            tail = bufs2[buf][pl.ds(j * 16, 16)]
            ge = cbase + j * 16 + lanes
            m = jnp.logical_and(ge >= lo, ge < hi)
            rel = a & 1023
            head = a >> 10
            tlv = plsc.load_gather(tl_v, [head])
            tlv = jnp.where(m, tlv, 0.0)
            for w in range(W):
                cv = plsc.load_gather(combo_v, [wvecs[w], rel])
                plsc.addupdate_scatter(neigh_v, [wvecs[w], tail], cv * tlv)

    def _batch(b, _):
        lo = jnp.clip(_range_at(b), ebase, eend)
        hi = jnp.clip(_range_at(b + 1), ebase, eend)
        start = lo & ~15
        nch = (hi - start + CHUNK - 1) >> 12

        @pl.when(nch > 0)
        def _():
            _issue(start, 0, 0)
        bvec = jnp.full((16,), b, jnp.int32)
        insb = [plsc.load_gather(ins_v, [wvecs[w], bvec]) for w in range(W)]

        def _cstep(i):
            for w in range(W):
                combo_v[w, pl.ds(i * 16, 16)] = (
                    rp_v[w, pl.ds(i * 16, 16)] * insb[w])

        @plsc.parallel_loop(0, (REL - 16 + 15) // 16, 1, unroll=4)
        def _cbody(i):
            _cstep(i)
        _last = REL - 16
        for w in range(W):
            combo_v[w, pl.ds(_last, 16)] = (
                rp_v[w, pl.ds(_last, 16)] * insb[w])

        def _pair(p, _):
            c0 = 2 * p
            _wait(0)

            @pl.when(c0 + 1 < nch)
            def _():
                _issue(start, c0 + 1, 1)
            _process(0, start + c0 * CHUNK, lo, hi)

            @pl.when(c0 + 1 < nch)
            def _():
                _wait(1)

                @pl.when(c0 + 2 < nch)
                def _():
                    _issue(start, c0 + 2, 0)
                _process(1, start + (c0 + 1) * CHUNK, lo, hi)
            return 0
        lax.fori_loop(0, (nch + 1) >> 1, _pair, 0)
        return 0

    lax.fori_loop(0, B, _batch, 0)

    pltpu.sync_copy(neigh_v, out_hbm.at[cid, pl.ds(fbase, W), :])


def _edge_sweep(pk1, pk2, tl_flat, rp_fm, ins_fm, rngs):
    f = pl.kernel(
        _edge_body,
        out_type=jax.ShapeDtypeStruct((NC, H, N), jnp.float32),
        mesh=_sc_mesh(),
        scratch_types=[
            pltpu.VMEM((W, REL), jnp.float32),
            pltpu.VMEM((W, REL), jnp.float32),
            pltpu.VMEM((W, B), jnp.float32),
            pltpu.VMEM((N,), jnp.float32),
            pltpu.VMEM((32,), jnp.int32),
            pltpu.VMEM((W, N), jnp.float32),
            pltpu.VMEM((CHUNK,), jnp.int32),
            pltpu.VMEM((CHUNK,), jnp.int32),
            pltpu.VMEM((CHUNK,), jnp.int32),
            pltpu.VMEM((CHUNK,), jnp.int32),
            pltpu.SemaphoreType.DMA,
            pltpu.SemaphoreType.DMA,
        ],
        compiler_params=pltpu.CompilerParams(needs_layout_passes=False),
    )
    return f(pk1, pk2, tl_flat, rp_fm, ins_fm, rngs)


def _tc_a_body(we_ref, q_ref, mb_ref, wx_ref, wh_ref, bl_ref, wq_ref, bq_ref,
               watt_ref, qmask_ref, rel_ref, lng_ref, lnb_ref, wrel_ref,
               brel_ref, rows_ref, went_ref, bent_ref,
               ht_ref, ins_ref, rp_ref, ent0_ref, xs_ref, hs_ref, xz_ref,
               sem):
    f32 = jnp.float32

    def _issue(i, _):
        b = i // LQ
        l = i % LQ
        idx = q_ref[b, l]
        pltpu.make_async_copy(we_ref.at[pl.ds(idx, 1), :],
                              xs_ref.at[l, pl.ds(b, 1), :], sem).start()
        return 0
    lax.fori_loop(0, B * LQ, _issue, 0)

    def _drain(i, _):
        pltpu.make_async_copy(we_ref.at[pl.ds(0, 1), :],
                              xs_ref.at[0, pl.ds(0, 1), :], sem).wait()
        return 0
    lax.fori_loop(0, B * LQ, _drain, 0)

    wh = wh_ref[...]
    bl = bl_ref[...]
    xz_ref[...] = jnp.dot(
        xs_ref[...].reshape(LQ * B, -1), wx_ref[...],
        preferred_element_type=f32).reshape(LQ, B, 4 * H)

    def step(t, carry):
        h, c = carry
        mt = mb_ref[t]
        z = (xz_ref[t]
             + jnp.dot(h, wh, preferred_element_type=f32) + bl[None, :])
        i = jax.nn.sigmoid(z[:, 0:H])
        fg = jax.nn.sigmoid(z[:, H:2 * H])
        g = jnp.tanh(z[:, 2 * H:3 * H])
        o = jax.nn.sigmoid(z[:, 3 * H:4 * H])
        cn = fg * c + i * g
        hn = o * jnp.tanh(cn)
        h2 = mt * hn + (1.0 - mt) * h
        c2 = mt * cn + (1.0 - mt) * c
        hs_ref[t] = h2
        return (h2, c2)

    h0 = jnp.zeros((B, H), f32)
    hT, _ = lax.fori_loop(0, LQ, step, (h0, h0))
    ht_ref[...] = hT

    hs = hs_ref[...]
    qmask = qmask_ref[...]
    for t in range(T):
        q_t = jnp.dot(hT, wq_ref[t], preferred_element_type=f32) + bq_ref[t][None, :]
        prod = hs * q_t[None, :, :] * watt_ref[t][None, None, :]
        logits = jnp.sum(prod, axis=2)
        logits = jnp.transpose(logits)
        logits = jnp.where(qmask > 0, logits, -1e20)
        mx = jnp.max(logits, axis=1, keepdims=True)
        ex = jnp.exp(logits - mx)
        attn = ex / jnp.sum(ex, axis=1, keepdims=True)
        attn_t = jnp.transpose(attn)
        ins_t = jnp.sum(attn_t[:, :, None] * hs, axis=0)
        ins_ref[t] = jnp.transpose(ins_t)

    rel = rel_ref[...]
    mu = jnp.mean(rel, axis=1, keepdims=True)
    var = jnp.mean((rel - mu) ** 2, axis=1, keepdims=True)
    ln = (rel - mu) / jnp.sqrt(var + 1e-5) * lng_ref[...][None, :] + lnb_ref[...][None, :]
    for t in range(T):
        rp_t = jnp.maximum(
            jnp.dot(ln, wrel_ref[t], preferred_element_type=f32) + brel_ref[t][None, :],
            0.0)
        rp_ref[t] = jnp.transpose(rp_t)

    ent0 = lax.dot_general(went_ref[...], rows_ref[...],
                           (((0,), (1,)), ((), ())),
                           preferred_element_type=f32)
    ent0_ref[...] = ent0 + bent_ref[...][:, None]


def _tc_a(question, mb, qmask, params, rows):
    out_shape = [
        jax.ShapeDtypeStruct((B, H), jnp.float32),
        jax.ShapeDtypeStruct((T, H, B), jnp.float32),
        jax.ShapeDtypeStruct((T, H, REL), jnp.float32),
        jax.ShapeDtypeStruct((H, N), jnp.float32),
    ]
    wdim = params["word_emb"].shape[1]
    nin = 18
    in_specs = [pl.BlockSpec(memory_space=pl.ANY),
                pl.BlockSpec(memory_space=pltpu.SMEM)]
    in_specs += [pl.BlockSpec(memory_space=pltpu.VMEM) for _ in range(nin - 2)]
    return pl.pallas_call(
        _tc_a_body,
        out_shape=out_shape,
        in_specs=in_specs,
        scratch_shapes=[pltpu.VMEM((LQ, B, wdim), jnp.float32),
                        pltpu.VMEM((LQ, B, H), jnp.float32),
                        pltpu.VMEM((LQ, B, 4 * H), jnp.float32),
                        pltpu.SemaphoreType.DMA],
    )(params["word_emb"], question, mb, params["Wx"], params["Wh"],
      params["b_lstm"], params["Wq"], params["bq"], params["w_att"], qmask,
      params["rel_emb"], params["ln_g"], params["ln_b"], params["W_rel"],
      params["b_rel"], rows, params["W_ent"], params["b_ent"])


def _node_update(neigh_ref, ent_ref, w1_ref, w2_ref, be_ref):
    f32 = jnp.float32
    nf = neigh_ref[0] + neigh_ref[1]
    z = (lax.dot_general(w1_ref[...], ent_ref[...], (((0,), (0,)), ((), ())),
                         preferred_element_type=f32)
         + lax.dot_general(w2_ref[...], nf, (((0,), (0,)), ((), ())),
                           preferred_element_type=f32)
         + be_ref[...][:, None])
    return jnp.maximum(z, 0.0) + ent_ref[...]


def _tc_cmid_body(neigh_ref, ent_ref, w1_ref, w2_ref, be_ref, ws_ref, bs_ref,
                  mask_ref, entnew_ref, tl_ref):
    ent_new = _node_update(neigh_ref, ent_ref, w1_ref, w2_ref, be_ref)
    entnew_ref[...] = ent_new
    s = jnp.sum(ent_new * ws_ref[...], axis=0, keepdims=True)
    s = s + bs_ref[...][:, None]
    rows = [s[0:1, b * M:(b + 1) * M] for b in range(B)]
    sb = jnp.concatenate(rows, axis=0)
    sb = jnp.where(mask_ref[...] > 0, sb, -1e20)
    mx = jnp.max(sb, axis=1, keepdims=True)
    ex = jnp.exp(sb - mx)
    tl_ref[...] = ex / jnp.sum(ex, axis=1, keepdims=True)


def _tc_cmid(neigh2, ent_fm, w1, w2, be, ws, bs, entity_mask):
    out_shape = [
        jax.ShapeDtypeStruct((H, N), jnp.float32),
        jax.ShapeDtypeStruct((B, M), jnp.float32),
    ]
    return pl.pallas_call(_tc_cmid_body, out_shape=out_shape)(
        neigh2, ent_fm, w1, w2, be, ws, bs, entity_mask)


def _tc_clast_body(neigh_ref, ent_ref, w1_ref, w2_ref, be_ref, wp_ref,
                   bp_ref, ht_ref, mask_ref, out_ref):
    f32 = jnp.float32
    ent_new = _node_update(neigh_ref, ent_ref, w1_ref, w2_ref, be_ref)
    ep = (lax.dot_general(wp_ref[...], ent_new, (((0,), (0,)), ((), ())),
                          preferred_element_type=f32)
          + bp_ref[...][:, None])
    big = jnp.dot(ht_ref[...], ep, preferred_element_type=f32)
    rows = [big[b:b + 1, b * M:(b + 1) * M] for b in range(B)]
    sc = jnp.concatenate(rows, axis=0)
    mask = mask_ref[...]
    out_ref[...] = mask * sc + (1.0 - mask) * -1e20


def _tc_clast(neigh2, ent_fm, w1, w2, be, wp, bp, ht, entity_mask):
    return pl.pallas_call(
        _tc_clast_body,
        out_shape=jax.ShapeDtypeStruct((B, M), jnp.float32),
    )(neigh2, ent_fm, w1, w2, be, wp, bp, ht, entity_mask)


def kernel(question, question_mask, topic_label, candidate_entity, entity_mask,
           batch_ids, batch_relations, edge_index, params):
    f32 = jnp.float32
    mb = jnp.swapaxes(question_mask, 0, 1)[:, :, None]

    idx = jnp.concatenate(
        [candidate_entity.reshape(-1).astype(jnp.int32),
         jnp.arange(GPAD - N, dtype=jnp.int32)])
    rows = _ent_gather(params["ent_emb"], idx)[:N]

    head = edge_index[0].astype(jnp.int32)
    tail = edge_index[1].astype(jnp.int32)
    zpad = jnp.zeros((CHUNK,), jnp.int32)
    pk1 = jnp.concatenate(
        [batch_relations.astype(jnp.int32) | (head << 10), zpad])
    pk2 = jnp.concatenate([tail, zpad])
    rngs = jnp.zeros((32,), jnp.int32).at[:B + 1].set(
        jnp.searchsorted(batch_ids, jnp.arange(B + 1)).astype(jnp.int32))

    hT, ins_fm, rp_fm, ent_fm = _tc_a(question.astype(jnp.int32), mb,
                                      question_mask, params, rows)

    tl = topic_label.reshape(-1).astype(f32)
    for t in range(T):
        neigh2 = _edge_sweep(pk1, pk2, tl, rp_fm[t], ins_fm[t], rngs)
        if t + 1 < T:
            ent_fm, tl_bm = _tc_cmid(neigh2, ent_fm,
                                     params["W_e"][t][:H],
                                     params["W_e"][t][H:],
                                     params["b_e"][t],
                                     params["w_score"][t],
                                     params["b_score"][t], entity_mask)
            tl = tl_bm.reshape(-1)
        else:
            return _tc_clast(neigh2, ent_fm, params["W_e"][t][:H],
                             params["W_e"][t][H:], params["b_e"][t],
                             params["W_proj"], params["b_proj"], hT,
                             entity_mask)
